# baseline (speedup 1.0000x reference)
import functools

import jax
import jax.numpy as jnp
from jax import lax
from jax.experimental import pallas as pl
from jax.experimental.pallas import tpu as pltpu
from jax.experimental.pallas import tpu_sc as plsc

_EMB = 1024
_ROWS = 8192
_NC = 2
_NS = 16
_NW = _NC * _NS
_RPW = _ROWS // _NW
_SPM_ROWS = 64
_SPM_HALF = _SPM_ROWS // 2
_CHUNK = 16
_NCHUNK = (_RPW - _SPM_ROWS) // _CHUNK
_NBUF = 4


@functools.partial(
    pl.kernel,
    mesh=plsc.VectorSubcoreMesh(core_axis_name="c", subcore_axis_name="s"),
    out_type=jax.ShapeDtypeStruct((_ROWS, _EMB), jnp.float32),
    scratch_types=(
        [pltpu.VMEM((_CHUNK, _EMB), jnp.float32)] * _NBUF
        + [pltpu.SemaphoreType.DMA] * (2 * _NBUF)
        + [pltpu.VMEM_SHARED((_NS, _SPM_ROWS, _EMB), jnp.float32)]
        + [pltpu.SemaphoreType.DMA] * 4
    ),
)
def _sc_row_copy(table_hbm, out_hbm, *refs):
    bufs = refs[:_NBUF]
    isems = refs[_NBUF:2 * _NBUF]
    osems = refs[2 * _NBUF:3 * _NBUF]
    spm = refs[3 * _NBUF]
    spm_sems = refs[3 * _NBUF + 1:]
    cid = lax.axis_index("c")
    sid = lax.axis_index("s")
    wid = sid * _NC + cid
    base = wid * _RPW
    spm_base = base + _NCHUNK * _CHUNK
    spm_reads = [
        pltpu.async_copy(
            table_hbm.at[pl.ds(spm_base + h * _SPM_HALF, _SPM_HALF)],
            spm.at[sid, pl.ds(h * _SPM_HALF, _SPM_HALF)], spm_sems[h])
        for h in range(2)
    ]
    spm_writes = [None, None]
    reads = [None] * _NBUF
    writes = [None] * _NBUF
    for i in range(_NBUF - 1):
        reads[i] = pltpu.async_copy(
            table_hbm.at[pl.ds(base + i * _CHUNK, _CHUNK)], bufs[i], isems[i])
    for i in range(_NCHUNK):
        b = i % _NBUF
        j = i + _NBUF - 1
        if j < _NCHUNK:
            jb = j % _NBUF
            if writes[jb] is not None:
                writes[jb].wait()
            reads[jb] = pltpu.async_copy(
                table_hbm.at[pl.ds(base + j * _CHUNK, _CHUNK)], bufs[jb],
                isems[jb])
        reads[b].wait()
        writes[b] = pltpu.async_copy(
            bufs[b], out_hbm.at[pl.ds(base + i * _CHUNK, _CHUNK)], osems[b])
        if i == _NCHUNK // 3 or i == (2 * _NCHUNK) // 3:
            h = 0 if i == _NCHUNK // 3 else 1
            spm_reads[h].wait()
            spm_writes[h] = pltpu.async_copy(
                spm.at[sid, pl.ds(h * _SPM_HALF, _SPM_HALF)],
                out_hbm.at[pl.ds(spm_base + h * _SPM_HALF, _SPM_HALF)],
                spm_sems[2 + h])
    for b in range(_NBUF):
        if writes[b] is not None:
            writes[b].wait()
    spm_writes[0].wait()
    spm_writes[1].wait()


def kernel(x, table):
    del x
    return _sc_row_copy(table)[None]


# speedup vs baseline: 1.0062x; 1.0062x over previous
import functools

import jax
import jax.numpy as jnp
from jax import lax
from jax.experimental import pallas as pl
from jax.experimental.pallas import tpu as pltpu
from jax.experimental.pallas import tpu_sc as plsc

_EMB = 1024
_ROWS = 8192
_NC = 2
_NS = 16
_NW = _NC * _NS
_RPW = _ROWS // _NW
_SPM_ROWS = 64
_SPM_HALF = _SPM_ROWS // 2
_CHUNK = 16
_NCHUNK = (_RPW - _SPM_ROWS) // _CHUNK
_NBUF = 4


@functools.partial(
    pl.kernel,
    mesh=plsc.VectorSubcoreMesh(core_axis_name="c", subcore_axis_name="s"),
    out_type=jax.ShapeDtypeStruct((_ROWS, _EMB), jnp.float32),
    scratch_types=(
        [pltpu.VMEM((_CHUNK, _EMB), jnp.float32)] * _NBUF
        + [pltpu.SemaphoreType.DMA] * (2 * _NBUF)
        + [pltpu.VMEM_SHARED((_NS, _SPM_ROWS, _EMB), jnp.float32)]
        + [pltpu.SemaphoreType.DMA] * 4
    ),
)
def _sc_row_copy(table_hbm, out_hbm, *refs):
    bufs = refs[:_NBUF]
    isems = refs[_NBUF:2 * _NBUF]
    osems = refs[2 * _NBUF:3 * _NBUF]
    spm = refs[3 * _NBUF]
    spm_sems = refs[3 * _NBUF + 1:]
    cid = lax.axis_index("c")
    sid = lax.axis_index("s")
    wid = sid * _NC + cid
    base = wid * _RPW
    spm_base = base + _NCHUNK * _CHUNK
    spm_reads = [
        pltpu.async_copy(
            table_hbm.at[pl.ds(spm_base, _SPM_ROWS)], spm.at[sid], spm_sems[0])
    ]
    spm_writes = [None]
    reads = [None] * _NBUF
    writes = [None] * _NBUF
    for i in range(_NBUF - 1):
        reads[i] = pltpu.async_copy(
            table_hbm.at[pl.ds(base + i * _CHUNK, _CHUNK)], bufs[i], isems[i])
    for i in range(_NCHUNK):
        b = i % _NBUF
        j = i + _NBUF - 1
        if j < _NCHUNK:
            jb = j % _NBUF
            if writes[jb] is not None:
                writes[jb].wait()
            reads[jb] = pltpu.async_copy(
                table_hbm.at[pl.ds(base + j * _CHUNK, _CHUNK)], bufs[jb],
                isems[jb])
        reads[b].wait()
        writes[b] = pltpu.async_copy(
            bufs[b], out_hbm.at[pl.ds(base + i * _CHUNK, _CHUNK)], osems[b])
        if i == _NCHUNK // 2:
            spm_reads[0].wait()
            spm_writes[0] = pltpu.async_copy(
                spm.at[sid], out_hbm.at[pl.ds(spm_base, _SPM_ROWS)],
                spm_sems[2])
    for b in range(_NBUF):
        if writes[b] is not None:
            writes[b].wait()
    spm_writes[0].wait()


def kernel(x, table):
    del x
    return _sc_row_copy(table)[None]


# speedup vs baseline: 1.0081x; 1.0019x over previous
import functools

import jax
import jax.numpy as jnp
from jax import lax
from jax.experimental import pallas as pl
from jax.experimental.pallas import tpu as pltpu
from jax.experimental.pallas import tpu_sc as plsc

_EMB = 1024
_ROWS = 8192
_NC = 2
_NS = 16
_NW = _NC * _NS
_RPW = _ROWS // _NW
_CHUNK = 16
_NCHUNK = _RPW // _CHUNK
_NBUF = 7


@functools.partial(
    pl.kernel,
    mesh=plsc.VectorSubcoreMesh(core_axis_name="c", subcore_axis_name="s"),
    out_type=jax.ShapeDtypeStruct((_ROWS, _EMB), jnp.float32),
    scratch_types=(
        [pltpu.VMEM((_CHUNK, _EMB), jnp.float32)] * _NBUF
        + [pltpu.SemaphoreType.DMA] * (2 * _NBUF)
    ),
)
def _sc_row_copy(table_hbm, out_hbm, *refs):
    bufs = refs[:_NBUF]
    isems = refs[_NBUF:2 * _NBUF]
    osems = refs[2 * _NBUF:]
    wid = lax.axis_index("s") * _NC + lax.axis_index("c")
    base = wid * _RPW
    reads = [None] * _NBUF
    writes = [None] * _NBUF
    for i in range(_NBUF - 1):
        reads[i] = pltpu.async_copy(
            table_hbm.at[pl.ds(base + i * _CHUNK, _CHUNK)], bufs[i], isems[i])
    for i in range(_NCHUNK):
        b = i % _NBUF
        j = i + _NBUF - 1
        if j < _NCHUNK:
            jb = j % _NBUF
            if writes[jb] is not None:
                writes[jb].wait()
            reads[jb] = pltpu.async_copy(
                table_hbm.at[pl.ds(base + j * _CHUNK, _CHUNK)], bufs[jb],
                isems[jb])
        reads[b].wait()
        writes[b] = pltpu.async_copy(
            bufs[b], out_hbm.at[pl.ds(base + i * _CHUNK, _CHUNK)], osems[b])
    for b in range(_NBUF):
        if writes[b] is not None:
            writes[b].wait()


def kernel(x, table):
    del x
    return _sc_row_copy(table)[None]
